# baseline (speedup 1.0000x reference)
import functools

import jax
import jax.numpy as jnp
from jax import lax
from jax.experimental import pallas as pl
from jax.experimental.pallas import tpu as pltpu
from jax.experimental.pallas import tpu_sc as plsc

EPS = 1e-5

N = 10000
E = 160000
NC, NS = 2, 16
NW = NC * NS
NPAD = 10112
RPS = NPAD // NS
EPW = E // NW
CH = 128
G = 3
GC = G * CH
NG = EPW // GC
TAIL = EPW - NG * GC
GA = 3
GCA = GA * CH
NGA = EPW // GCA
ATAIL = EPW - NGA * GCA

def _mesh():
    return plsc.VectorSubcoreMesh(core_axis_name="c", subcore_axis_name="s",
                                  num_cores=NC, num_subcores=NS)


@jax.jit
def _sc_degree(dst, onehot, zeros128):
    @functools.partial(
        pl.kernel,
        out_type=jax.ShapeDtypeStruct((NC, NPAD, 128), jnp.float32),
        mesh=_mesh(),
        scratch_types=[
            pltpu.VMEM_SHARED((NPAD, 128), jnp.float32),
            pltpu.VMEM((G, CH), jnp.int32),
            pltpu.VMEM((1, TAIL), jnp.int32),
            pltpu.VMEM((CH, 128), jnp.float32),
            pltpu.SemaphoreType.DMA,
        ],
    )
    def deg_kernel(dst_hbm, oh_hbm, z_hbm, out_hbm, acc_sh, idx_v, idxt_v, oh_v,
                   sem):
        c = lax.axis_index("c")
        s = lax.axis_index("s")
        wid = s * NC + c
        base = wid * EPW
        pltpu.sync_copy(z_hbm, acc_sh.at[pl.ds(s * RPS, RPS)])
        pltpu.sync_copy(oh_hbm, oh_v)
        plsc.subcore_barrier()

        @pl.loop(0, NG)
        def _(g):
            off = base + g * GC
            for j in range(G):
                pltpu.sync_copy(dst_hbm.at[pl.ds(off + j * CH, CH)], idx_v.at[j])
            hs = [pltpu.async_copy(oh_v, acc_sh.at[idx_v.at[j]], sem, add=True)
                  for j in range(G)]
            for h in hs:
                h.wait()

        pltpu.sync_copy(dst_hbm.at[pl.ds(base + NG * GC, TAIL)], idxt_v.at[0])
        pltpu.sync_copy(oh_v.at[pl.ds(0, TAIL)], acc_sh.at[idxt_v.at[0]], add=True)

        plsc.subcore_barrier()
        pltpu.sync_copy(acc_sh.at[pl.ds(s * RPS, RPS)],
                        out_hbm.at[c, pl.ds(s * RPS, RPS)])

    return deg_kernel(dst, onehot, zeros128)


@jax.jit
def _sc_aggregate(hp, src, dst, zeros128):
    @functools.partial(
        pl.kernel,
        out_type=jax.ShapeDtypeStruct((NC, NPAD, 128), jnp.float32),
        mesh=_mesh(),
        scratch_types=[
            pltpu.VMEM_SHARED((NPAD, 128), jnp.float32),
            pltpu.VMEM((GA, CH), jnp.int32),
            pltpu.VMEM((1, TAIL), jnp.int32),
            pltpu.VMEM((1, TAIL), jnp.int32),
            pltpu.VMEM((GCA, 128), jnp.float32),
            pltpu.SemaphoreType.DMA,
            pltpu.SemaphoreType.DMA,
        ],
    )
    def agg_kernel(hp_hbm, src_hbm, dst_hbm, z_hbm, out_hbm,
                   acc_sh, idx_v, sit_v, dit_v, rows_v,
                   sem_g, sem_s):
        c = lax.axis_index("c")
        s = lax.axis_index("s")
        wid = s * NC + c
        base = wid * EPW
        pltpu.sync_copy(z_hbm, acc_sh.at[pl.ds(s * RPS, RPS)])
        plsc.subcore_barrier()

        @pl.loop(0, NGA)
        def _(g):
            off = base + g * GCA
            for j in range(GA):
                pltpu.sync_copy(src_hbm.at[pl.ds(off + j * CH, CH)], idx_v.at[j])
            hs = [pltpu.async_copy(hp_hbm.at[idx_v.at[j]],
                                   rows_v.at[pl.ds(j * CH, CH)], sem_g)
                  for j in range(GA)]
            for h in hs:
                h.wait()
            for j in range(GA):
                pltpu.sync_copy(dst_hbm.at[pl.ds(off + j * CH, CH)], idx_v.at[j])
            ss = [pltpu.async_copy(rows_v.at[pl.ds(j * CH, CH)],
                                   acc_sh.at[idx_v.at[j]], sem_s, add=True)
                  for j in range(GA)]
            for h in ss:
                h.wait()

        t0 = base + NGA * GCA
        pltpu.sync_copy(src_hbm.at[pl.ds(t0, ATAIL)], sit_v.at[0])
        pltpu.sync_copy(dst_hbm.at[pl.ds(t0, ATAIL)], dit_v.at[0])
        pltpu.async_copy(hp_hbm.at[sit_v.at[0]],
                         rows_v.at[pl.ds(0, ATAIL)], sem_g).wait()
        pltpu.sync_copy(rows_v.at[pl.ds(0, ATAIL)], acc_sh.at[dit_v.at[0]],
                        add=True)

        plsc.subcore_barrier()
        pltpu.sync_copy(acc_sh.at[pl.ds(s * RPS, RPS)],
                        out_hbm.at[c, pl.ds(s * RPS, RPS)])

    return agg_kernel(hp, src, dst, zeros128)


@jax.jit
def _sc_edge_gather(h, src, dst):
    @functools.partial(
        pl.kernel,
        out_type=(jax.ShapeDtypeStruct((E, 128), jnp.float32),
                  jax.ShapeDtypeStruct((E, 128), jnp.float32)),
        mesh=_mesh(),
        scratch_types=[
            pltpu.VMEM((GC,), jnp.int32),
            pltpu.VMEM((GC,), jnp.int32),
            pltpu.VMEM((TAIL,), jnp.int32),
            pltpu.VMEM((TAIL,), jnp.int32),
            pltpu.VMEM((GC, 128), jnp.float32),
            pltpu.VMEM((GC, 128), jnp.float32),
            pltpu.VMEM((TAIL, 128), jnp.float32),
            pltpu.VMEM((TAIL, 128), jnp.float32),
            pltpu.SemaphoreType.DMA,
            pltpu.SemaphoreType.DMA,
            pltpu.SemaphoreType.DMA,
            pltpu.SemaphoreType.DMA,
        ],
    )
    def gather_kernel(h_hbm, src_hbm, dst_hbm, gs_hbm, gd_hbm,
                      si_v, di_v, sit_v, dit_v, rs_v, rd_v, rst_v, rdt_v,
                      sem0, sem1, sem2, sem3):
        c = lax.axis_index("c")
        s = lax.axis_index("s")
        wid = s * NC + c
        base = wid * EPW

        @pl.loop(0, NG)
        def _(g):
            off = base + g * GC
            pltpu.sync_copy(src_hbm.at[pl.ds(off, GC)], si_v)
            pltpu.sync_copy(dst_hbm.at[pl.ds(off, GC)], di_v)
            hs = [pltpu.async_copy(h_hbm.at[si_v.at[pl.ds(j * CH, CH)]],
                                   rs_v.at[pl.ds(j * CH, CH)], sem0)
                  for j in range(G)]
            hd = [pltpu.async_copy(h_hbm.at[di_v.at[pl.ds(j * CH, CH)]],
                                   rd_v.at[pl.ds(j * CH, CH)], sem1)
                  for j in range(G)]
            for h in hs:
                h.wait()
            ws = pltpu.async_copy(rs_v, gs_hbm.at[pl.ds(off, GC)], sem2)
            for h in hd:
                h.wait()
            wd = pltpu.async_copy(rd_v, gd_hbm.at[pl.ds(off, GC)], sem3)
            ws.wait()
            wd.wait()

        t0 = base + NG * GC
        pltpu.sync_copy(src_hbm.at[pl.ds(t0, TAIL)], sit_v)
        pltpu.sync_copy(dst_hbm.at[pl.ds(t0, TAIL)], dit_v)
        cs = pltpu.async_copy(h_hbm.at[sit_v], rst_v, sem0)
        cd = pltpu.async_copy(h_hbm.at[dit_v], rdt_v, sem1)
        cs.wait()
        pltpu.sync_copy(rst_v, gs_hbm.at[pl.ds(t0, TAIL)])
        cd.wait()
        pltpu.sync_copy(rdt_v, gd_hbm.at[pl.ds(t0, TAIL)])

    return gather_kernel(h, src, dst)


def _node_a_body(x_ref, w0t_ref, degp_ref, hp_ref, dinv_ref):
    deg = degp_ref[0, 0:N, 0:1] + degp_ref[1, 0:N, 0:1] + 1.0
    dinv = lax.rsqrt(jnp.maximum(deg, 1e-12))
    dinv_ref[...] = dinv
    h = jnp.dot(x_ref[...], w0t_ref[...], preferred_element_type=jnp.float32)
    hp_ref[...] = dinv * h


def _tc_node_a(x, w0t, degp):
    return pl.pallas_call(
        _node_a_body,
        out_shape=(jax.ShapeDtypeStruct((N, 128), jnp.float32),
                   jax.ShapeDtypeStruct((N, 1), jnp.float32)),
    )(x, w0t, degp)


def _node_b_body(accp_ref, hp_ref, dinv_ref, b0_ref, w1_ref, b1_ref, w2t_ref,
                 x1_ref, hp2_ref):
    acc = accp_ref[0, 0:N, :] + accp_ref[1, 0:N, :]
    dinv = dinv_ref[...]
    conv1 = dinv * (acc + hp_ref[...]) + b0_ref[...]
    m = jnp.mean(conv1, axis=0, keepdims=True)
    v = jnp.mean((conv1 - m) * (conv1 - m), axis=0, keepdims=True)
    x1 = jnp.maximum((conv1 - m) * lax.rsqrt(v + EPS) * w1_ref[...]
                     + b1_ref[...], 0.0)
    x1_ref[...] = x1
    h2 = jnp.dot(x1, w2t_ref[...], preferred_element_type=jnp.float32)
    hp2_ref[...] = dinv * h2


def _tc_node_b(accp, hp, dinv, b0, w1, b1, w2t):
    return pl.pallas_call(
        _node_b_body,
        out_shape=(jax.ShapeDtypeStruct((N, 128), jnp.float32),
                   jax.ShapeDtypeStruct((N, 128), jnp.float32)),
    )(accp, hp, dinv, b0, w1, b1, w2t)


def _node_c_body(accp_ref, hp2_ref, dinv_ref, b2_ref, w3_ref, b3_ref, x1_ref,
                 h_ref):
    acc = accp_ref[0, 0:N, :] + accp_ref[1, 0:N, :]
    conv2 = dinv_ref[...] * (acc + hp2_ref[...]) + b2_ref[...]
    m = jnp.mean(conv2, axis=0, keepdims=True)
    v = jnp.mean((conv2 - m) * (conv2 - m), axis=0, keepdims=True)
    bn = (conv2 - m) * lax.rsqrt(v + EPS) * w3_ref[...] + b3_ref[...]
    h_ref[...] = jnp.maximum(bn + x1_ref[...], 0.0)


def _tc_node_c(accp, hp2, dinv, b2, w3, b3, x1):
    return pl.pallas_call(
        _node_c_body,
        out_shape=jax.ShapeDtypeStruct((N, 128), jnp.float32),
    )(accp, hp2, dinv, b2, w3, b3, x1)


BE = 2000
NEB = E // BE


def _mlp1_body(gs_ref, gd_ref, w4a_ref, w4b_ref, b4_ref, t1_ref, st_ref):
    gs = gs_ref[...].astype(jnp.bfloat16)
    gd = gd_ref[...].astype(jnp.bfloat16)
    t1 = (jnp.dot(gs, w4a_ref[...], preferred_element_type=jnp.float32)
          + jnp.dot(gd, w4b_ref[...], preferred_element_type=jnp.float32)
          + b4_ref[...])
    t1_ref[...] = t1.astype(jnp.bfloat16)

    @pl.when(pl.program_id(0) == 0)
    def _():
        st_ref[...] = jnp.zeros_like(st_ref)

    st_ref[0:1, :] += jnp.sum(t1, axis=0, keepdims=True)
    st_ref[1:2, :] += jnp.sum(t1 * t1, axis=0, keepdims=True)


def _tc_mlp1(gs, gd, w4at, w4bt, b4):
    return pl.pallas_call(
        _mlp1_body,
        grid=(NEB,),
        in_specs=[
            pl.BlockSpec((BE, 128), lambda i: (i, 0)),
            pl.BlockSpec((BE, 128), lambda i: (i, 0)),
            pl.BlockSpec((128, 256), lambda i: (0, 0)),
            pl.BlockSpec((128, 256), lambda i: (0, 0)),
            pl.BlockSpec((1, 256), lambda i: (0, 0)),
        ],
        out_specs=(pl.BlockSpec((BE, 256), lambda i: (i, 0)),
                   pl.BlockSpec((8, 256), lambda i: (0, 0))),
        out_shape=(jax.ShapeDtypeStruct((E, 256), jnp.bfloat16),
                   jax.ShapeDtypeStruct((8, 256), jnp.float32)),
    )(gs, gd, w4at, w4bt, b4)


def _bn_coeffs(st_ref, w_ref, b_ref):
    m = st_ref[0:1, :] * (1.0 / E)
    msq = st_ref[1:2, :] * (1.0 / E)
    v = msq - m * m
    scale = w_ref[...] * lax.rsqrt(v + EPS)
    shift = b_ref[...] - m * scale
    return scale, shift


def _mlp2_body(t1_ref, st1_ref, w5_ref, b5_ref, w6t_ref, b6_ref,
               t2_ref, st2_ref):
    scale, shift = _bn_coeffs(st1_ref, w5_ref, b5_ref)
    y = jnp.maximum(t1_ref[...].astype(jnp.float32) * scale + shift, 0.0)
    t2 = (jnp.dot(y.astype(jnp.bfloat16), w6t_ref[...],
                  preferred_element_type=jnp.float32) + b6_ref[...])
    t2_ref[...] = t2.astype(jnp.bfloat16)

    @pl.when(pl.program_id(0) == 0)
    def _():
        st2_ref[...] = jnp.zeros_like(st2_ref)

    st2_ref[0:1, :] += jnp.sum(t2, axis=0, keepdims=True)
    st2_ref[1:2, :] += jnp.sum(t2 * t2, axis=0, keepdims=True)


def _tc_mlp2(t1, st1, w5, b5, w6t, b6):
    return pl.pallas_call(
        _mlp2_body,
        grid=(NEB,),
        in_specs=[
            pl.BlockSpec((BE, 256), lambda i: (i, 0)),
            pl.BlockSpec((8, 256), lambda i: (0, 0)),
            pl.BlockSpec((1, 256), lambda i: (0, 0)),
            pl.BlockSpec((1, 256), lambda i: (0, 0)),
            pl.BlockSpec((256, 256), lambda i: (0, 0)),
            pl.BlockSpec((1, 256), lambda i: (0, 0)),
        ],
        out_specs=(pl.BlockSpec((BE, 256), lambda i: (i, 0)),
                   pl.BlockSpec((8, 256), lambda i: (0, 0))),
        out_shape=(jax.ShapeDtypeStruct((E, 256), jnp.bfloat16),
                   jax.ShapeDtypeStruct((8, 256), jnp.float32)),
    )(t1, st1, w5, b5, w6t, b6)


def _mlp3_body(t2_ref, st2_ref, w7_ref, b7_ref, w8t_ref, b8_ref, out_ref):
    scale, shift = _bn_coeffs(st2_ref, w7_ref, b7_ref)
    y = jnp.maximum(t2_ref[...].astype(jnp.float32) * scale + shift, 0.0)
    out_ref[...] = (jnp.dot(y.astype(jnp.bfloat16), w8t_ref[...],
                            preferred_element_type=jnp.float32) + b8_ref[...])


def _tc_mlp3(t2, st2, w7, b7, w8t, b8):
    return pl.pallas_call(
        _mlp3_body,
        grid=(NEB,),
        in_specs=[
            pl.BlockSpec((BE, 256), lambda i: (i, 0)),
            pl.BlockSpec((8, 256), lambda i: (0, 0)),
            pl.BlockSpec((1, 256), lambda i: (0, 0)),
            pl.BlockSpec((1, 256), lambda i: (0, 0)),
            pl.BlockSpec((256, 128), lambda i: (0, 0)),
            pl.BlockSpec((1, 128), lambda i: (0, 0)),
        ],
        out_specs=pl.BlockSpec((BE, 128), lambda i: (i, 0)),
        out_shape=jax.ShapeDtypeStruct((E, 128), jnp.float32),
    )(t2, st2, w7, b7, w8t, b8)


def kernel(x, edge_index, w0, b0, w1, b1, w2, b2, w3, b3, w4, b4, w5, b5,
           w6, b6, w7, b7, w8, b8):
    src = edge_index[0].astype(jnp.int32)
    dst = edge_index[1].astype(jnp.int32)

    onehot = jnp.tile(
        (jnp.arange(128, dtype=jnp.int32) == 0).astype(jnp.float32)[None, :],
        (CH, 1))
    zeros128 = jnp.zeros((RPS, 128), jnp.float32)

    row = lambda a: a.reshape(1, -1)

    degp = _sc_degree(dst, onehot, zeros128)
    hp1, dinv = _tc_node_a(x, w0.T, degp)
    accp1 = _sc_aggregate(hp1, src, dst, zeros128)
    x1, hp2 = _tc_node_b(accp1, hp1, dinv, row(b0), row(w1), row(b1), w2.T)
    accp2 = _sc_aggregate(hp2, src, dst, zeros128)
    hfin = _tc_node_c(accp2, hp2, dinv, row(b2), row(w3), row(b3), x1)
    gs, gd = _sc_edge_gather(hfin, src, dst)
    w4a, w4b = w4[:, :128], w4[:, 128:]
    bf = lambda a: a.astype(jnp.bfloat16)
    t1, st1 = _tc_mlp1(gs, gd, bf(w4a.T), bf(w4b.T), row(b4))
    t2, st2 = _tc_mlp2(t1, st1, row(w5), row(b5), bf(w6.T), row(b6))
    return _tc_mlp3(t2, st2, row(w7), row(b7), bf(w8.T), row(b8))


# speedup vs baseline: 1.0205x; 1.0205x over previous
import functools

import jax
import jax.numpy as jnp
from jax import lax
from jax.experimental import pallas as pl
from jax.experimental.pallas import tpu as pltpu
from jax.experimental.pallas import tpu_sc as plsc

EPS = 1e-5

N = 10000
E = 160000
NC, NS = 2, 16
NW = NC * NS
NPAD = 10240
RPS = NPAD // NS
EPW = E // NW
CH = 128
G = 3
GC = G * CH
NG = EPW // GC
TAIL = EPW - NG * GC
GA = 2
GCA = GA * CH
NGA = EPW // GCA
ATAIL = EPW - NGA * GCA

def _mesh():
    return plsc.VectorSubcoreMesh(core_axis_name="c", subcore_axis_name="s",
                                  num_cores=NC, num_subcores=NS)


@jax.jit
def _sc_degree(dst, onehot, zeros128):
    @functools.partial(
        pl.kernel,
        out_type=jax.ShapeDtypeStruct((NC, NPAD, 128), jnp.float32),
        mesh=_mesh(),
        scratch_types=[
            pltpu.VMEM_SHARED((NPAD, 128), jnp.float32),
            pltpu.VMEM((G, CH), jnp.int32),
            pltpu.VMEM((1, TAIL), jnp.int32),
            pltpu.VMEM((CH, 128), jnp.float32),
            pltpu.SemaphoreType.DMA,
        ],
    )
    def deg_kernel(dst_hbm, oh_hbm, z_hbm, out_hbm, acc_sh, idx_v, idxt_v, oh_v,
                   sem):
        c = lax.axis_index("c")
        s = lax.axis_index("s")
        wid = s * NC + c
        base = wid * EPW
        pltpu.sync_copy(z_hbm, acc_sh.at[pl.ds(s * RPS, RPS)])
        pltpu.sync_copy(oh_hbm, oh_v)
        plsc.subcore_barrier()

        @pl.loop(0, NG)
        def _(g):
            off = base + g * GC
            for j in range(G):
                pltpu.sync_copy(dst_hbm.at[pl.ds(off + j * CH, CH)], idx_v.at[j])
            hs = [pltpu.async_copy(oh_v, acc_sh.at[idx_v.at[j]], sem, add=True)
                  for j in range(G)]
            for h in hs:
                h.wait()

        pltpu.sync_copy(dst_hbm.at[pl.ds(base + NG * GC, TAIL)], idxt_v.at[0])
        pltpu.sync_copy(oh_v.at[pl.ds(0, TAIL)], acc_sh.at[idxt_v.at[0]], add=True)

        plsc.subcore_barrier()
        pltpu.sync_copy(acc_sh.at[pl.ds(s * RPS, RPS)],
                        out_hbm.at[c, pl.ds(s * RPS, RPS)])

    return deg_kernel(dst, onehot, zeros128)


@jax.jit
def _sc_aggregate(hp, src, dst, zeros128):
    @functools.partial(
        pl.kernel,
        out_type=jax.ShapeDtypeStruct((NC, NPAD, 128), jnp.float32),
        mesh=_mesh(),
        scratch_types=[
            pltpu.VMEM_SHARED((NPAD, 128), jnp.float32),
            pltpu.VMEM((GCA,), jnp.int32),
            pltpu.VMEM((GA + 1, CH), jnp.int32),
            pltpu.VMEM((1, TAIL), jnp.int32),
            pltpu.VMEM((1, TAIL), jnp.int32),
            pltpu.VMEM((GCA, 128), jnp.float32),
            pltpu.VMEM((TAIL, 128), jnp.float32),
            pltpu.SemaphoreType.DMA,
            pltpu.SemaphoreType.DMA,
        ],
    )
    def agg_kernel(hp_hbm, src_hbm, dst_hbm, z_hbm, out_hbm,
                   acc_sh, si_v, di_v, sit_v, dit_v, rows_v, rowst_v,
                   sem_g, sem_s):
        c = lax.axis_index("c")
        s = lax.axis_index("s")
        wid = s * NC + c
        base = wid * EPW
        pltpu.sync_copy(z_hbm, acc_sh.at[pl.ds(s * RPS, RPS)])
        plsc.subcore_barrier()

        @pl.loop(0, NGA)
        def _(g):
            off = base + g * GCA
            pltpu.sync_copy(src_hbm.at[pl.ds(off, GCA)], si_v)
            for j in range(GA):
                pltpu.sync_copy(dst_hbm.at[pl.ds(off + j * CH, CH)], di_v.at[j])
            hs = [pltpu.async_copy(hp_hbm.at[si_v.at[pl.ds(j * CH, CH)]],
                                   rows_v.at[pl.ds(j * CH, CH)], sem_g)
                  for j in range(GA)]
            for h in hs:
                h.wait()
            ss = [pltpu.async_copy(rows_v.at[pl.ds(j * CH, CH)],
                                   acc_sh.at[di_v.at[j]], sem_s, add=True)
                  for j in range(GA)]
            for h in ss:
                h.wait()

        t0 = base + NGA * GCA
        pltpu.sync_copy(src_hbm.at[pl.ds(t0, CH)], si_v.at[pl.ds(0, CH)])
        pltpu.sync_copy(dst_hbm.at[pl.ds(t0, CH)], di_v.at[GA])
        pltpu.async_copy(hp_hbm.at[si_v.at[pl.ds(0, CH)]],
                         rows_v.at[pl.ds(0, CH)], sem_g).wait()
        pltpu.sync_copy(rows_v.at[pl.ds(0, CH)], acc_sh.at[di_v.at[GA]],
                        add=True)
        t1 = t0 + CH
        pltpu.sync_copy(src_hbm.at[pl.ds(t1, TAIL)], sit_v.at[0])
        pltpu.sync_copy(dst_hbm.at[pl.ds(t1, TAIL)], dit_v.at[0])
        pltpu.async_copy(hp_hbm.at[sit_v.at[0]], rowst_v, sem_g).wait()
        pltpu.sync_copy(rowst_v, acc_sh.at[dit_v.at[0]], add=True)

        plsc.subcore_barrier()
        pltpu.sync_copy(acc_sh.at[pl.ds(s * RPS, RPS)],
                        out_hbm.at[c, pl.ds(s * RPS, RPS)])

    return agg_kernel(hp, src, dst, zeros128)


KCH = 4
EC = E // KCH
EPWC = 1248
NGC = EPWC // GC
TAILC = EPWC - NGC * GC
EXTRA = EC - NW * EPWC


@jax.jit
def _sc_edge_gather(h, src, dst):
    @functools.partial(
        pl.kernel,
        out_type=(jax.ShapeDtypeStruct((EC, 128), jnp.float32),
                  jax.ShapeDtypeStruct((EC, 128), jnp.float32)),
        mesh=_mesh(),
        scratch_types=[
            pltpu.VMEM((GC,), jnp.int32),
            pltpu.VMEM((GC,), jnp.int32),
            pltpu.VMEM((TAILC,), jnp.int32),
            pltpu.VMEM((TAILC,), jnp.int32),
            pltpu.VMEM((GC, 128), jnp.float32),
            pltpu.VMEM((GC, 128), jnp.float32),
            pltpu.VMEM((TAILC, 128), jnp.float32),
            pltpu.VMEM((TAILC, 128), jnp.float32),
            pltpu.SemaphoreType.DMA,
            pltpu.SemaphoreType.DMA,
            pltpu.SemaphoreType.DMA,
            pltpu.SemaphoreType.DMA,
        ],
    )
    def gather_kernel(h_hbm, src_hbm, dst_hbm, gs_hbm, gd_hbm,
                      si_v, di_v, sit_v, dit_v, rs_v, rd_v, rst_v, rdt_v,
                      sem0, sem1, sem2, sem3):
        c = lax.axis_index("c")
        s = lax.axis_index("s")
        wid = s * NC + c
        base = wid * EPWC

        @pl.loop(0, NGC)
        def _(g):
            off = base + g * GC
            pltpu.sync_copy(src_hbm.at[pl.ds(off, GC)], si_v)
            pltpu.sync_copy(dst_hbm.at[pl.ds(off, GC)], di_v)
            hs = [pltpu.async_copy(h_hbm.at[si_v.at[pl.ds(j * CH, CH)]],
                                   rs_v.at[pl.ds(j * CH, CH)], sem0)
                  for j in range(G)]
            hd = [pltpu.async_copy(h_hbm.at[di_v.at[pl.ds(j * CH, CH)]],
                                   rd_v.at[pl.ds(j * CH, CH)], sem1)
                  for j in range(G)]
            for h in hs:
                h.wait()
            ws = pltpu.async_copy(rs_v, gs_hbm.at[pl.ds(off, GC)], sem2)
            for h in hd:
                h.wait()
            wd = pltpu.async_copy(rd_v, gd_hbm.at[pl.ds(off, GC)], sem3)
            ws.wait()
            wd.wait()

        t0 = base + NGC * GC
        pltpu.sync_copy(src_hbm.at[pl.ds(t0, TAILC)], sit_v)
        pltpu.sync_copy(dst_hbm.at[pl.ds(t0, TAILC)], dit_v)
        cs = pltpu.async_copy(h_hbm.at[sit_v], rst_v, sem0)
        cd = pltpu.async_copy(h_hbm.at[dit_v], rdt_v, sem1)
        cs.wait()
        pltpu.sync_copy(rst_v, gs_hbm.at[pl.ds(t0, TAILC)])
        cd.wait()
        pltpu.sync_copy(rdt_v, gd_hbm.at[pl.ds(t0, TAILC)])

        @pl.when(jnp.logical_and(c == NC - 1, s == NS - 1))
        def _():
            x0 = NW * EPWC
            pltpu.sync_copy(src_hbm.at[pl.ds(x0, EXTRA)],
                            sit_v.at[pl.ds(0, EXTRA)])
            pltpu.sync_copy(dst_hbm.at[pl.ds(x0, EXTRA)],
                            dit_v.at[pl.ds(0, EXTRA)])
            xs = pltpu.async_copy(h_hbm.at[sit_v.at[pl.ds(0, EXTRA)]],
                                  rst_v.at[pl.ds(0, EXTRA)], sem0)
            xd = pltpu.async_copy(h_hbm.at[dit_v.at[pl.ds(0, EXTRA)]],
                                  rdt_v.at[pl.ds(0, EXTRA)], sem1)
            xs.wait()
            pltpu.sync_copy(rst_v.at[pl.ds(0, EXTRA)],
                            gs_hbm.at[pl.ds(x0, EXTRA)])
            xd.wait()
            pltpu.sync_copy(rdt_v.at[pl.ds(0, EXTRA)],
                            gd_hbm.at[pl.ds(x0, EXTRA)])

    return gather_kernel(h, src, dst)


def _node_a_body(x_ref, w0t_ref, degp_ref, hp_ref, dinv_ref):
    deg = degp_ref[0, 0:N, 0:1] + degp_ref[1, 0:N, 0:1] + 1.0
    dinv = lax.rsqrt(jnp.maximum(deg, 1e-12))
    dinv_ref[...] = dinv
    h = jnp.dot(x_ref[...], w0t_ref[...], preferred_element_type=jnp.float32)
    hp_ref[...] = dinv * h


def _tc_node_a(x, w0t, degp):
    return pl.pallas_call(
        _node_a_body,
        out_shape=(jax.ShapeDtypeStruct((N, 128), jnp.float32),
                   jax.ShapeDtypeStruct((N, 1), jnp.float32)),
    )(x, w0t, degp)


def _node_b_body(accp_ref, hp_ref, dinv_ref, b0_ref, w1_ref, b1_ref, w2t_ref,
                 x1_ref, hp2_ref):
    acc = accp_ref[0, 0:N, :] + accp_ref[1, 0:N, :]
    dinv = dinv_ref[...]
    conv1 = dinv * (acc + hp_ref[...]) + b0_ref[...]
    m = jnp.mean(conv1, axis=0, keepdims=True)
    v = jnp.mean((conv1 - m) * (conv1 - m), axis=0, keepdims=True)
    x1 = jnp.maximum((conv1 - m) * lax.rsqrt(v + EPS) * w1_ref[...]
                     + b1_ref[...], 0.0)
    x1_ref[...] = x1
    h2 = jnp.dot(x1, w2t_ref[...], preferred_element_type=jnp.float32)
    hp2_ref[...] = dinv * h2


def _tc_node_b(accp, hp, dinv, b0, w1, b1, w2t):
    return pl.pallas_call(
        _node_b_body,
        out_shape=(jax.ShapeDtypeStruct((N, 128), jnp.float32),
                   jax.ShapeDtypeStruct((N, 128), jnp.float32)),
    )(accp, hp, dinv, b0, w1, b1, w2t)


def _node_c_body(accp_ref, hp2_ref, dinv_ref, b2_ref, w3_ref, b3_ref, x1_ref,
                 h_ref):
    acc = accp_ref[0, 0:N, :] + accp_ref[1, 0:N, :]
    conv2 = dinv_ref[...] * (acc + hp2_ref[...]) + b2_ref[...]
    m = jnp.mean(conv2, axis=0, keepdims=True)
    v = jnp.mean((conv2 - m) * (conv2 - m), axis=0, keepdims=True)
    bn = (conv2 - m) * lax.rsqrt(v + EPS) * w3_ref[...] + b3_ref[...]
    h_ref[...] = jnp.maximum(bn + x1_ref[...], 0.0)


def _tc_node_c(accp, hp2, dinv, b2, w3, b3, x1):
    return pl.pallas_call(
        _node_c_body,
        out_shape=jax.ShapeDtypeStruct((N, 128), jnp.float32),
    )(accp, hp2, dinv, b2, w3, b3, x1)


BE = 2000
NEBC = EC // BE


def _mlp1_body(gs_ref, gd_ref, w4a_ref, w4b_ref, b4_ref, t1_ref, st_ref):
    gs = gs_ref[...].astype(jnp.bfloat16)
    gd = gd_ref[...].astype(jnp.bfloat16)
    t1 = (jnp.dot(gs, w4a_ref[...], preferred_element_type=jnp.float32)
          + jnp.dot(gd, w4b_ref[...], preferred_element_type=jnp.float32)
          + b4_ref[...])
    t1_ref[...] = t1.astype(jnp.bfloat16)

    @pl.when(pl.program_id(0) == 0)
    def _():
        st_ref[...] = jnp.zeros_like(st_ref)

    st_ref[0:1, :] += jnp.sum(t1, axis=0, keepdims=True)
    st_ref[1:2, :] += jnp.sum(t1 * t1, axis=0, keepdims=True)


def _tc_mlp1(gs, gd, w4at, w4bt, b4):
    return pl.pallas_call(
        _mlp1_body,
        grid=(NEBC,),
        in_specs=[
            pl.BlockSpec((BE, 128), lambda i: (i, 0)),
            pl.BlockSpec((BE, 128), lambda i: (i, 0)),
            pl.BlockSpec((128, 256), lambda i: (0, 0)),
            pl.BlockSpec((128, 256), lambda i: (0, 0)),
            pl.BlockSpec((1, 256), lambda i: (0, 0)),
        ],
        out_specs=(pl.BlockSpec((BE, 256), lambda i: (i, 0)),
                   pl.BlockSpec((8, 256), lambda i: (0, 0))),
        out_shape=(jax.ShapeDtypeStruct((EC, 256), jnp.bfloat16),
                   jax.ShapeDtypeStruct((8, 256), jnp.float32)),
    )(gs, gd, w4at, w4bt, b4)


def _bn_coeffs(st_ref, w_ref, b_ref):
    ssum = st_ref[0:1, :]
    ssq = st_ref[1:2, :]
    for k in range(1, KCH):
        ssum = ssum + st_ref[8 * k:8 * k + 1, :]
        ssq = ssq + st_ref[8 * k + 1:8 * k + 2, :]
    m = ssum * (1.0 / E)
    msq = ssq * (1.0 / E)
    v = msq - m * m
    scale = w_ref[...] * lax.rsqrt(v + EPS)
    shift = b_ref[...] - m * scale
    return scale, shift


def _mlp2_body(t1_ref, st1_ref, w5_ref, b5_ref, w6t_ref, b6_ref,
               t2_ref, st2_ref):
    scale, shift = _bn_coeffs(st1_ref, w5_ref, b5_ref)
    y = jnp.maximum(t1_ref[...].astype(jnp.float32) * scale + shift, 0.0)
    t2 = (jnp.dot(y.astype(jnp.bfloat16), w6t_ref[...],
                  preferred_element_type=jnp.float32) + b6_ref[...])
    t2_ref[...] = t2.astype(jnp.bfloat16)

    @pl.when(pl.program_id(0) == 0)
    def _():
        st2_ref[...] = jnp.zeros_like(st2_ref)

    st2_ref[0:1, :] += jnp.sum(t2, axis=0, keepdims=True)
    st2_ref[1:2, :] += jnp.sum(t2 * t2, axis=0, keepdims=True)


def _tc_mlp2(t1, st1, w5, b5, w6t, b6):
    return pl.pallas_call(
        _mlp2_body,
        grid=(NEBC,),
        in_specs=[
            pl.BlockSpec((BE, 256), lambda i: (i, 0)),
            pl.BlockSpec((8 * KCH, 256), lambda i: (0, 0)),
            pl.BlockSpec((1, 256), lambda i: (0, 0)),
            pl.BlockSpec((1, 256), lambda i: (0, 0)),
            pl.BlockSpec((256, 256), lambda i: (0, 0)),
            pl.BlockSpec((1, 256), lambda i: (0, 0)),
        ],
        out_specs=(pl.BlockSpec((BE, 256), lambda i: (i, 0)),
                   pl.BlockSpec((8, 256), lambda i: (0, 0))),
        out_shape=(jax.ShapeDtypeStruct((EC, 256), jnp.bfloat16),
                   jax.ShapeDtypeStruct((8, 256), jnp.float32)),
    )(t1, st1, w5, b5, w6t, b6)


def _mlp3_body(t2_ref, st2_ref, w7_ref, b7_ref, w8t_ref, b8_ref, out_ref):
    scale, shift = _bn_coeffs(st2_ref, w7_ref, b7_ref)
    y = jnp.maximum(t2_ref[...].astype(jnp.float32) * scale + shift, 0.0)
    out_ref[...] = (jnp.dot(y.astype(jnp.bfloat16), w8t_ref[...],
                            preferred_element_type=jnp.float32) + b8_ref[...])


def _mlp3_body_alias(t2_ref, st2_ref, w7_ref, b7_ref, w8t_ref, b8_ref,
                     prev_ref, out_ref):
    del prev_ref
    _mlp3_body(t2_ref, st2_ref, w7_ref, b7_ref, w8t_ref, b8_ref, out_ref)


def _tc_mlp3(t2, st2, w7, b7, w8t, b8, coff, prev):
    in_specs = [
        pl.BlockSpec((BE, 256), lambda i: (i, 0)),
        pl.BlockSpec((8 * KCH, 256), lambda i: (0, 0)),
        pl.BlockSpec((1, 256), lambda i: (0, 0)),
        pl.BlockSpec((1, 256), lambda i: (0, 0)),
        pl.BlockSpec((256, 128), lambda i: (0, 0)),
        pl.BlockSpec((1, 128), lambda i: (0, 0)),
    ]
    ins = [t2, st2, w7, b7, w8t, b8]
    body = _mlp3_body
    kwargs = {}
    if prev is not None:
        in_specs.append(pl.BlockSpec(memory_space=pl.ANY))
        ins.append(prev)
        body = _mlp3_body_alias
        kwargs = dict(input_output_aliases={6: 0})
    return pl.pallas_call(
        body,
        grid=(NEBC,),
        in_specs=in_specs,
        out_specs=pl.BlockSpec((BE, 128), lambda i: (i + coff, 0)),
        out_shape=jax.ShapeDtypeStruct((E, 128), jnp.float32),
        **kwargs,
    )(*ins)


def kernel(x, edge_index, w0, b0, w1, b1, w2, b2, w3, b3, w4, b4, w5, b5,
           w6, b6, w7, b7, w8, b8):
    src = edge_index[0].astype(jnp.int32)
    dst = edge_index[1].astype(jnp.int32)

    onehot = jnp.tile(
        (jnp.arange(128, dtype=jnp.int32) == 0).astype(jnp.float32)[None, :],
        (CH, 1))
    zeros128 = jnp.zeros((RPS, 128), jnp.float32)

    row = lambda a: a.reshape(1, -1)

    degp = _sc_degree(dst, onehot, zeros128)
    hp1, dinv = _tc_node_a(x, w0.T, degp)
    accp1 = _sc_aggregate(hp1, src, dst, zeros128)
    x1, hp2 = _tc_node_b(accp1, hp1, dinv, row(b0), row(w1), row(b1), w2.T)
    accp2 = _sc_aggregate(hp2, src, dst, zeros128)
    hfin = _tc_node_c(accp2, hp2, dinv, row(b2), row(w3), row(b3), x1)
    w4a, w4b = w4[:, :128], w4[:, 128:]
    bf = lambda a: a.astype(jnp.bfloat16)
    w4at, w4bt, b4r = bf(w4a.T), bf(w4b.T), row(b4)

    gathered = [_sc_edge_gather(hfin, src[c * EC:(c + 1) * EC],
                                dst[c * EC:(c + 1) * EC])
                for c in range(KCH)]
    t1s, st1s = [], []
    for gs_c, gd_c in gathered:
        t1_c, st1_c = _tc_mlp1(gs_c, gd_c, w4at, w4bt, b4r)
        t1s.append(t1_c)
        st1s.append(st1_c)
    st1 = jnp.concatenate(st1s, axis=0)
    t2s, st2s = [], []
    for t1_c in t1s:
        t2_c, st2_c = _tc_mlp2(t1_c, st1, row(w5), row(b5), bf(w6.T), row(b6))
        t2s.append(t2_c)
        st2s.append(st2_c)
    st2 = jnp.concatenate(st2s, axis=0)
    out = None
    for c, t2_c in enumerate(t2s):
        out = _tc_mlp3(t2_c, st2, row(w7), row(b7), bf(w8.T), row(b8),
                       c * NEBC, out)
    return out
